# Initial kernel scaffold; baseline (speedup 1.0000x reference)
import functools

import jax
import jax.numpy as jnp
from jax import lax
from jax.experimental import pallas as pl
from jax.experimental.pallas import tpu as pltpu
from jax.experimental.pallas import tpu_sc as plsc

NC = 2
NS = 16
NW = NC * NS


def _gather_rows_sc(x, idx, chunk):
    e, w = idx.shape[0], x.shape[1]
    per_w = e // NW
    n_ch = per_w // chunk
    mesh = plsc.VectorSubcoreMesh(core_axis_name="c", subcore_axis_name="s")

    @functools.partial(
        pl.kernel,
        out_type=jax.ShapeDtypeStruct((e, w), jnp.float32),
        mesh=mesh,
        scratch_types=[
            pltpu.VMEM((chunk,), jnp.int32),
            pltpu.VMEM((chunk, w), jnp.float32),
            pltpu.SemaphoreType.DMA,
        ],
    )
    def k(x_hbm, idx_hbm, out_hbm, idx_v, rows_v, sem):
        wid = lax.axis_index("s") * NC + lax.axis_index("c")
        base = wid * per_w

        def body(j, carry):
            off = base + j * chunk
            pltpu.sync_copy(idx_hbm.at[pl.ds(off, chunk)], idx_v)
            pltpu.async_copy(x_hbm.at[idx_v], rows_v, sem).wait()
            pltpu.sync_copy(rows_v, out_hbm.at[pl.ds(off, chunk)])
            return carry

        lax.fori_loop(0, n_ch, body, 0)

    return k(x, idx)


def _scatter_add_sc(msgs, idx, n_nodes, chunk):
    e, w = msgs.shape
    per_w = e // NW
    n_ch = per_w // chunk
    rows_per_tile = n_nodes // NS
    mesh = plsc.VectorSubcoreMesh(core_axis_name="c", subcore_axis_name="s")
    zeros = jnp.zeros((n_nodes, w), jnp.float32)

    @functools.partial(
        pl.kernel,
        out_type=jax.ShapeDtypeStruct((NC, n_nodes, w), jnp.float32),
        mesh=mesh,
        scratch_types=[
            pltpu.VMEM((chunk,), jnp.int32),
            pltpu.VMEM((chunk, w), jnp.float32),
            pltpu.VMEM_SHARED((n_nodes, w), jnp.float32),
            pltpu.SemaphoreType.DMA,
        ],
    )
    def k(msg_hbm, idx_hbm, zeros_hbm, out_hbm, idx_v, rows_v, acc_sh, sem):
        c = lax.axis_index("c")
        s = lax.axis_index("s")
        r0 = s * rows_per_tile
        pltpu.sync_copy(zeros_hbm.at[pl.ds(r0, rows_per_tile)],
                        acc_sh.at[pl.ds(r0, rows_per_tile)])
        plsc.subcore_barrier()

        base = (c * NS + s) * per_w

        def body(j, carry):
            off = base + j * chunk
            pltpu.sync_copy(idx_hbm.at[pl.ds(off, chunk)], idx_v)
            pltpu.sync_copy(msg_hbm.at[pl.ds(off, chunk)], rows_v)
            pltpu.sync_copy(rows_v, acc_sh.at[idx_v], add=True)
            return carry

        lax.fori_loop(0, n_ch, body, 0)
        plsc.subcore_barrier()
        pltpu.sync_copy(acc_sh.at[pl.ds(r0, rows_per_tile)],
                        out_hbm.at[c, pl.ds(r0, rows_per_tile)])

    return k(msgs, idx, zeros)


def _messages_tc(ea, xj, Wt, b0, blk):
    e, d_dim = ea.shape
    w = xj.shape[1]

    def body(ea_ref, xj_ref, wt_ref, b0_ref, out_ref):
        xj_b = xj_ref[...]
        ea_b = ea_ref[...]
        y = jnp.dot(xj_b, wt_ref[...], preferred_element_type=jnp.float32)
        acc = jnp.dot(xj_b, b0_ref[...], preferred_element_type=jnp.float32)
        for d in range(d_dim):
            acc = acc + ea_b[:, d:d + 1] * y[:, d * w:(d + 1) * w]
        out_ref[...] = acc

    return pl.pallas_call(
        body,
        grid=(e // blk,),
        in_specs=[
            pl.BlockSpec((blk, d_dim), lambda i: (i, 0)),
            pl.BlockSpec((blk, w), lambda i: (i, 0)),
            pl.BlockSpec((w, d_dim * w), lambda i: (0, 0)),
            pl.BlockSpec((w, w), lambda i: (0, 0)),
        ],
        out_specs=pl.BlockSpec((blk, w), lambda i: (i, 0)),
        out_shape=jax.ShapeDtypeStruct((e, w), jnp.float32),
    )(ea, xj, Wt, b0)


def kernel(x, senders, receivers, edge_attr, W_nn, b_nn):
    n_nodes, w = x.shape
    d_dim = edge_attr.shape[1]
    senders = senders.astype(jnp.int32)
    receivers = receivers.astype(jnp.int32)
    Wt = W_nn.reshape(d_dim, w, w).transpose(1, 0, 2).reshape(w, d_dim * w)
    b0 = b_nn.reshape(w, w)

    xj = _gather_rows_sc(x, senders, chunk=5000)
    msgs = _messages_tc(edge_attr, xj, Wt, b0, blk=4000)
    partials = _scatter_add_sc(msgs, receivers, n_nodes, chunk=5000)
    return partials[0] + partials[1]


# speedup vs baseline: 1.3064x; 1.3064x over previous
import functools

import jax
import jax.numpy as jnp
from jax import lax
from jax.experimental import pallas as pl
from jax.experimental.pallas import tpu as pltpu
from jax.experimental.pallas import tpu_sc as plsc

NC = 2
NS = 16
NW = NC * NS


def _gather_rows_sc(x, idx, chunk):
    e, w = idx.shape[0], x.shape[1]
    per_w = e // NW
    n_ch = per_w // chunk
    mesh = plsc.VectorSubcoreMesh(core_axis_name="c", subcore_axis_name="s")

    @functools.partial(
        pl.kernel,
        out_type=jax.ShapeDtypeStruct((e, w), jnp.float32),
        mesh=mesh,
        scratch_types=[
            pltpu.VMEM((chunk,), jnp.int32),
            pltpu.VMEM((chunk, w), jnp.float32),
            pltpu.SemaphoreType.DMA,
        ],
        compiler_params=pltpu.CompilerParams(use_tc_tiling_on_sc=False),
    )
    def k(x_hbm, idx_hbm, out_hbm, idx_v, rows_v, sem):
        wid = lax.axis_index("s") * NC + lax.axis_index("c")
        base = wid * per_w

        def body(j, carry):
            off = base + j * chunk
            pltpu.sync_copy(idx_hbm.at[pl.ds(off, chunk)], idx_v)
            pltpu.async_copy(x_hbm.at[idx_v], rows_v, sem).wait()
            pltpu.sync_copy(rows_v, out_hbm.at[pl.ds(off, chunk)])
            return carry

        lax.fori_loop(0, n_ch, body, 0)

    return k(x, idx)


def _scatter_add_sc(msgs, idx, n_nodes, chunk):
    e, w = msgs.shape
    per_w = e // NW
    n_ch = per_w // chunk
    rows_per_tile = n_nodes // NS
    mesh = plsc.VectorSubcoreMesh(core_axis_name="c", subcore_axis_name="s")
    zeros = jnp.zeros((n_nodes, w), jnp.float32)

    @functools.partial(
        pl.kernel,
        out_type=jax.ShapeDtypeStruct((NC, n_nodes, w), jnp.float32),
        mesh=mesh,
        scratch_types=[
            pltpu.VMEM((chunk,), jnp.int32),
            pltpu.VMEM((chunk, w), jnp.float32),
            pltpu.VMEM_SHARED((n_nodes, w), jnp.float32),
            pltpu.SemaphoreType.DMA,
        ],
        compiler_params=pltpu.CompilerParams(use_tc_tiling_on_sc=False),
    )
    def k(msg_hbm, idx_hbm, zeros_hbm, out_hbm, idx_v, rows_v, acc_sh, sem):
        c = lax.axis_index("c")
        s = lax.axis_index("s")
        r0 = s * rows_per_tile
        pltpu.sync_copy(zeros_hbm.at[pl.ds(r0, rows_per_tile)],
                        acc_sh.at[pl.ds(r0, rows_per_tile)])
        plsc.subcore_barrier()

        base = (c * NS + s) * per_w

        def body(j, carry):
            off = base + j * chunk
            pltpu.sync_copy(idx_hbm.at[pl.ds(off, chunk)], idx_v)
            pltpu.sync_copy(msg_hbm.at[pl.ds(off, chunk)], rows_v)
            pltpu.sync_copy(rows_v, acc_sh.at[idx_v], add=True)
            return carry

        lax.fori_loop(0, n_ch, body, 0)
        plsc.subcore_barrier()
        pltpu.sync_copy(acc_sh.at[pl.ds(r0, rows_per_tile)],
                        out_hbm.at[c, pl.ds(r0, rows_per_tile)])

    return k(msgs, idx, zeros)


def _messages_tc(ea, xj, Wt, b0, blk):
    e, d_dim = ea.shape
    w = xj.shape[1]

    def body(ea_ref, xj_ref, wt_ref, b0_ref, out_ref):
        xj_b = xj_ref[...]
        ea_b = ea_ref[...]
        y = jnp.dot(xj_b, wt_ref[...], preferred_element_type=jnp.float32)
        acc = jnp.dot(xj_b, b0_ref[...], preferred_element_type=jnp.float32)
        for d in range(d_dim):
            acc = acc + ea_b[:, d:d + 1] * y[:, d * w:(d + 1) * w]
        out_ref[...] = acc

    return pl.pallas_call(
        body,
        grid=(e // blk,),
        in_specs=[
            pl.BlockSpec((blk, d_dim), lambda i: (i, 0)),
            pl.BlockSpec((blk, w), lambda i: (i, 0)),
            pl.BlockSpec((w, d_dim * w), lambda i: (0, 0)),
            pl.BlockSpec((w, w), lambda i: (0, 0)),
        ],
        out_specs=pl.BlockSpec((blk, w), lambda i: (i, 0)),
        out_shape=jax.ShapeDtypeStruct((e, w), jnp.float32),
    )(ea, xj, Wt, b0)


def kernel(x, senders, receivers, edge_attr, W_nn, b_nn):
    n_nodes, w = x.shape
    d_dim = edge_attr.shape[1]
    senders = senders.astype(jnp.int32)
    receivers = receivers.astype(jnp.int32)
    Wt = W_nn.reshape(d_dim, w, w).transpose(1, 0, 2).reshape(w, d_dim * w)
    b0 = b_nn.reshape(w, w)

    xj = _gather_rows_sc(x, senders, chunk=5000)
    msgs = _messages_tc(edge_attr, xj, Wt, b0, blk=4000)
    partials = _scatter_add_sc(msgs, receivers, n_nodes, chunk=1000)
    return partials[0] + partials[1]


# speedup vs baseline: 3.8032x; 2.9112x over previous
import functools

import jax
import jax.numpy as jnp
from jax import lax
from jax.experimental import pallas as pl
from jax.experimental.pallas import tpu as pltpu
from jax.experimental.pallas import tpu_sc as plsc

NC = 2
NS = 16
NW = NC * NS


def _gather_rows_sc(x, idx, chunk):
    e, w = idx.shape[0], x.shape[1]
    per_w = e // NW
    n_ch = per_w // chunk
    mesh = plsc.VectorSubcoreMesh(core_axis_name="c", subcore_axis_name="s")

    @functools.partial(
        pl.kernel,
        out_type=jax.ShapeDtypeStruct((e, w), jnp.float32),
        mesh=mesh,
        scratch_types=[
            pltpu.VMEM((chunk,), jnp.int32),
            pltpu.VMEM((chunk, w), jnp.float32),
            pltpu.SemaphoreType.DMA,
        ],
        compiler_params=pltpu.CompilerParams(use_tc_tiling_on_sc=False),
    )
    def k(x_hbm, idx_hbm, out_hbm, idx_v, rows_v, sem):
        wid = lax.axis_index("s") * NC + lax.axis_index("c")
        base = wid * per_w

        def body(j, carry):
            off = base + j * chunk
            pltpu.sync_copy(idx_hbm.at[pl.ds(off, chunk)], idx_v)
            pltpu.async_copy(x_hbm.at[idx_v], rows_v, sem).wait()
            pltpu.sync_copy(rows_v, out_hbm.at[pl.ds(off, chunk)])
            return carry

        lax.fori_loop(0, n_ch, body, 0)

    return k(x, idx)


def _scatter_add_sc(msgs, idx, n_nodes, chunk):
    e, w = msgs.shape
    per_w = e // NW
    n_ch = per_w // chunk
    rows_per_tile = n_nodes // NS
    mesh = plsc.VectorSubcoreMesh(core_axis_name="c", subcore_axis_name="s")
    zeros = jnp.zeros((n_nodes, w), jnp.float32)

    @functools.partial(
        pl.kernel,
        out_type=jax.ShapeDtypeStruct((NC, n_nodes, w), jnp.float32),
        mesh=mesh,
        scratch_types=[
            pltpu.VMEM((chunk,), jnp.int32),
            pltpu.VMEM((chunk, w), jnp.float32),
            pltpu.VMEM_SHARED((n_nodes, w), jnp.float32),
            pltpu.SemaphoreType.DMA,
        ],
        compiler_params=pltpu.CompilerParams(use_tc_tiling_on_sc=False),
    )
    def k(msg_hbm, idx_hbm, zeros_hbm, out_hbm, idx_v, rows_v, acc_sh, sem):
        c = lax.axis_index("c")
        s = lax.axis_index("s")
        r0 = s * rows_per_tile
        pltpu.sync_copy(zeros_hbm.at[pl.ds(r0, rows_per_tile)],
                        acc_sh.at[pl.ds(r0, rows_per_tile)])
        plsc.subcore_barrier()

        base = (c * NS + s) * per_w

        def body(j, carry):
            off = base + j * chunk
            pltpu.sync_copy(idx_hbm.at[pl.ds(off, chunk)], idx_v)
            pltpu.sync_copy(msg_hbm.at[pl.ds(off, chunk)], rows_v)
            pltpu.sync_copy(rows_v, acc_sh.at[idx_v], add=True)
            return carry

        lax.fori_loop(0, n_ch, body, 0)
        plsc.subcore_barrier()
        pltpu.sync_copy(acc_sh.at[pl.ds(r0, rows_per_tile)],
                        out_hbm.at[c, pl.ds(r0, rows_per_tile)])

    return k(msgs, idx, zeros)


def _messages_tc(ea, xj, R, T, W2, b0, blk):
    e, d_dim = ea.shape
    w = xj.shape[1]

    def body(ea_ref, xj_ref, r_ref, t_ref, w2_ref, b0_ref, out_ref):
        ea_b = ea_ref[...]
        xj_b = xj_ref[...]
        z = (jnp.dot(ea_b, r_ref[...], preferred_element_type=jnp.float32)
             * jnp.dot(xj_b, t_ref[...], preferred_element_type=jnp.float32))
        out_ref[...] = (
            jnp.dot(z, w2_ref[...], preferred_element_type=jnp.float32)
            + jnp.dot(xj_b, b0_ref[...], preferred_element_type=jnp.float32))

    return pl.pallas_call(
        body,
        grid=(e // blk,),
        in_specs=[
            pl.BlockSpec((blk, d_dim), lambda i: (i, 0)),
            pl.BlockSpec((blk, w), lambda i: (i, 0)),
            pl.BlockSpec((d_dim, d_dim * w), lambda i: (0, 0)),
            pl.BlockSpec((w, d_dim * w), lambda i: (0, 0)),
            pl.BlockSpec((d_dim * w, w), lambda i: (0, 0)),
            pl.BlockSpec((w, w), lambda i: (0, 0)),
        ],
        out_specs=pl.BlockSpec((blk, w), lambda i: (i, 0)),
        out_shape=jax.ShapeDtypeStruct((e, w), jnp.float32),
    )(ea, xj, R, T, W2, b0)


def kernel(x, senders, receivers, edge_attr, W_nn, b_nn):
    n_nodes, w = x.shape
    d_dim = edge_attr.shape[1]
    senders = senders.astype(jnp.int32)
    receivers = receivers.astype(jnp.int32)
    eye = jnp.eye(w, dtype=jnp.float32)
    R = jnp.repeat(eye, w, axis=1)
    T = jnp.tile(eye, (1, d_dim))
    W2 = W_nn.reshape(d_dim * w, w)
    b0 = b_nn.reshape(w, w)

    xj = _gather_rows_sc(x, senders, chunk=5000)
    msgs = _messages_tc(edge_attr, xj, R, T, W2, b0, blk=4000)
    partials = _scatter_add_sc(msgs, receivers, n_nodes, chunk=1000)
    return partials[0] + partials[1]


# speedup vs baseline: 5.9995x; 1.5775x over previous
import functools

import jax
import jax.numpy as jnp
from jax import lax
from jax.experimental import pallas as pl
from jax.experimental.pallas import tpu as pltpu
from jax.experimental.pallas import tpu_sc as plsc

NC = 2
NS = 16
NW = NC * NS


def _gather_rows_sc(x, idx, chunk):
    e, w = idx.shape[0], x.shape[1]
    per_w = e // NW
    n_ch = per_w // chunk
    mesh = plsc.VectorSubcoreMesh(core_axis_name="c", subcore_axis_name="s")

    @functools.partial(
        pl.kernel,
        out_type=jax.ShapeDtypeStruct((e, w), jnp.float32),
        mesh=mesh,
        scratch_types=[
            pltpu.VMEM((chunk,), jnp.int32),
            pltpu.VMEM((chunk, w), jnp.float32),
            pltpu.SemaphoreType.DMA,
        ],
        compiler_params=pltpu.CompilerParams(use_tc_tiling_on_sc=False),
    )
    def k(x_hbm, idx_hbm, out_hbm, idx_v, rows_v, sem):
        wid = lax.axis_index("s") * NC + lax.axis_index("c")
        base = wid * per_w

        def body(j, carry):
            off = base + j * chunk
            pltpu.sync_copy(idx_hbm.at[pl.ds(off, chunk)], idx_v)
            pltpu.async_copy(x_hbm.at[idx_v], rows_v, sem).wait()
            pltpu.sync_copy(rows_v, out_hbm.at[pl.ds(off, chunk)])
            return carry

        lax.fori_loop(0, n_ch, body, 0)

    return k(x, idx)


def _scatter_add_sc(msgs, idx, n_nodes, chunk):
    e, w = msgs.shape
    per_w = e // NW
    n_ch = per_w // chunk
    rows_per_tile = n_nodes // NS
    mesh = plsc.VectorSubcoreMesh(core_axis_name="c", subcore_axis_name="s")
    zeros = jnp.zeros((n_nodes, w), jnp.float32)

    @functools.partial(
        pl.kernel,
        out_type=jax.ShapeDtypeStruct((NC, n_nodes, w), jnp.float32),
        mesh=mesh,
        scratch_types=[
            pltpu.VMEM((chunk,), jnp.int32),
            pltpu.VMEM((chunk, w), jnp.float32),
            pltpu.VMEM_SHARED((n_nodes, w), jnp.float32),
            pltpu.SemaphoreType.DMA,
        ],
        compiler_params=pltpu.CompilerParams(use_tc_tiling_on_sc=False),
    )
    def k(msg_hbm, idx_hbm, zeros_hbm, out_hbm, idx_v, rows_v, acc_sh, sem):
        c = lax.axis_index("c")
        s = lax.axis_index("s")
        r0 = s * rows_per_tile
        pltpu.sync_copy(zeros_hbm.at[pl.ds(r0, rows_per_tile)],
                        acc_sh.at[pl.ds(r0, rows_per_tile)])
        plsc.subcore_barrier()

        base = (c * NS + s) * per_w

        def body(j, carry):
            off = base + j * chunk
            pltpu.sync_copy(idx_hbm.at[pl.ds(off, chunk)], idx_v)
            pltpu.sync_copy(msg_hbm.at[pl.ds(off, chunk)], rows_v)
            pltpu.sync_copy(rows_v, acc_sh.at[idx_v], add=True)
            return carry

        lax.fori_loop(0, n_ch, body, 0)
        plsc.subcore_barrier()
        pltpu.sync_copy(acc_sh.at[pl.ds(r0, rows_per_tile)],
                        out_hbm.at[c, pl.ds(r0, rows_per_tile)])

    return k(msgs, idx, zeros)


def _messages_tc(ea8, xj8, Rp, Tp, W2p, B0p, blk8):
    e8, pw = ea8.shape

    def body(ea_ref, xj_ref, r_ref, t_ref, w2_ref, b0_ref, out_ref):
        ea_b = ea_ref[...]
        xj_b = xj_ref[...]
        z = (jnp.dot(ea_b, r_ref[...], preferred_element_type=jnp.float32)
             * jnp.dot(xj_b, t_ref[...], preferred_element_type=jnp.float32))
        out_ref[...] = (
            jnp.dot(z, w2_ref[...], preferred_element_type=jnp.float32)
            + jnp.dot(xj_b, b0_ref[...], preferred_element_type=jnp.float32))

    zw = Rp.shape[1]
    return pl.pallas_call(
        body,
        grid=(e8 // blk8,),
        in_specs=[
            pl.BlockSpec((blk8, pw), lambda i: (i, 0)),
            pl.BlockSpec((blk8, pw), lambda i: (i, 0)),
            pl.BlockSpec((pw, zw), lambda i: (0, 0)),
            pl.BlockSpec((pw, zw), lambda i: (0, 0)),
            pl.BlockSpec((zw, pw), lambda i: (0, 0)),
            pl.BlockSpec((pw, pw), lambda i: (0, 0)),
        ],
        out_specs=pl.BlockSpec((blk8, pw), lambda i: (i, 0)),
        out_shape=jax.ShapeDtypeStruct((e8, pw), jnp.float32),
    )(ea8, xj8, Rp, Tp, W2p, B0p)


def kernel(x, senders, receivers, edge_attr, W_nn, b_nn):
    n_nodes, w = x.shape
    d_dim = edge_attr.shape[1]
    senders = senders.astype(jnp.int32)
    receivers = receivers.astype(jnp.int32)
    eye = jnp.eye(w, dtype=jnp.float32)
    i8 = jnp.eye(8, dtype=jnp.float32)
    R = jnp.repeat(eye, w, axis=1)
    T = jnp.tile(eye, (1, d_dim))
    W2 = W_nn.reshape(d_dim * w, w)
    b0 = b_nn.reshape(w, w)
    Rp = jnp.kron(i8, R)
    Tp = jnp.kron(i8, T)
    W2p = jnp.kron(i8, W2)
    B0p = jnp.kron(i8, b0)

    e = senders.shape[0]
    ea8 = edge_attr.reshape(e // 8, 8 * d_dim)
    xj = _gather_rows_sc(x, senders, chunk=5000)
    xj8 = xj.reshape(e // 8, 8 * w)
    msgs8 = _messages_tc(ea8, xj8, Rp, Tp, W2p, B0p, blk8=800)
    msgs = msgs8.reshape(e, w)
    partials = _scatter_add_sc(msgs, receivers, n_nodes, chunk=1000)
    return partials[0] + partials[1]
